# baseline (speedup 1.0000x reference)
import functools

import jax
import jax.numpy as jnp
from jax import lax
from jax.experimental import pallas as pl
from jax.experimental.pallas import tpu as pltpu
from jax.experimental.pallas import tpu_sc as plsc

N = 10000
E = 320000
IN_DIM = 128
HID_DIM = 32

NC = 2
NS = 16
NW = NC * NS
LANES = 16

E_PER_W = E // NW
CHUNK = 80
N_CHUNKS = E_PER_W // CHUNK
NBUF = 4
ROWS_PER_TILE = N // NS
NP = 10240

_mesh = plsc.VectorSubcoreMesh(core_axis_name="c", subcore_axis_name="s")
_sc_params = pltpu.CompilerParams(
    needs_layout_passes=False, use_tc_tiling_on_sc=False)


def _zero_1d(ref, total):
  def body(i, _):
    ref[pl.ds(i * LANES, LANES)] = jnp.zeros((LANES,), ref.dtype)
    return 0
  lax.fori_loop(0, total // LANES, body, 0, unroll=8)


@functools.partial(
    pl.kernel,
    out_type=jax.ShapeDtypeStruct((NW * NP,), jnp.float32),
    mesh=_mesh,
    compiler_params=_sc_params,
    scratch_types=[
        pltpu.VMEM((N_CHUNKS, CHUNK), jnp.int32),
        pltpu.VMEM((NP,), jnp.float32),
    ],
)
def _sc_degree(col_hbm, out_hbm, col_v, acc_v):
  wid = lax.axis_index("s") * NC + lax.axis_index("c")
  pltpu.sync_copy(col_hbm.at[wid], col_v)
  _zero_1d(acc_v, NP)
  ones = jnp.ones((LANES,), jnp.float32)

  def body(k, _):
    for t in range(CHUNK // LANES):
      c = col_v[k, pl.ds(t * LANES, LANES)]
      plsc.addupdate_scatter(acc_v, [c], ones)
    return 0
  lax.fori_loop(0, N_CHUNKS, body, 0, unroll=2)
  pltpu.sync_copy(acc_v, out_hbm.at[pl.ds(wid * NP, NP)])


@functools.partial(
    pl.kernel,
    out_type=jax.ShapeDtypeStruct((NC, N, HID_DIM), jnp.float32),
    mesh=_mesh,
    compiler_params=_sc_params,
    scratch_types=[
        pltpu.VMEM((N_CHUNKS, CHUNK), jnp.int32),
        pltpu.VMEM((N_CHUNKS, CHUNK), jnp.int32),
        pltpu.VMEM((NBUF, CHUNK, HID_DIM), jnp.float32),
        pltpu.VMEM((ROWS_PER_TILE, HID_DIM), jnp.float32),
        pltpu.VMEM_SHARED((N, HID_DIM), jnp.float32),
        pltpu.SemaphoreType.DMA,
    ],
)
def _sc_agg1(row_hbm, col_hbm, y_hbm, out_hbm,
             ridx_v, cidx_v, rows_v, stage_v, agg_sh, sem):
  cid = lax.axis_index("c")
  sid = lax.axis_index("s")
  wid = sid * NC + cid

  def zbody(j, _):
    stage_v[j, pl.ds(0, LANES)] = jnp.zeros((LANES,), jnp.float32)
    stage_v[j, pl.ds(LANES, LANES)] = jnp.zeros((LANES,), jnp.float32)
    return 0
  lax.fori_loop(0, ROWS_PER_TILE, zbody, 0, unroll=8)
  pltpu.sync_copy(stage_v, agg_sh.at[pl.ds(sid * ROWS_PER_TILE, ROWS_PER_TILE)])

  pltpu.sync_copy(row_hbm.at[wid], ridx_v)
  pltpu.sync_copy(col_hbm.at[wid], cidx_v)
  plsc.subcore_barrier()

  for b in range(NBUF):
    pltpu.async_copy(y_hbm.at[ridx_v.at[b]], rows_v.at[b], sem)

  def body(k, _):
    b = lax.rem(k, NBUF)
    pltpu.make_async_copy(y_hbm.at[ridx_v.at[k]], rows_v.at[b], sem).wait()
    pltpu.sync_copy(rows_v.at[b], agg_sh.at[cidx_v.at[k]], add=True)
    nk = k + NBUF

    @pl.when(nk < N_CHUNKS)
    def _():
      pltpu.async_copy(y_hbm.at[ridx_v.at[nk]], rows_v.at[b], sem)
    return 0
  lax.fori_loop(0, N_CHUNKS, body, 0)
  plsc.subcore_barrier()

  pltpu.sync_copy(agg_sh.at[pl.ds(sid * ROWS_PER_TILE, ROWS_PER_TILE)], stage_v)
  pltpu.sync_copy(stage_v, out_hbm.at[cid, pl.ds(sid * ROWS_PER_TILE, ROWS_PER_TILE)])


@functools.partial(
    pl.kernel,
    out_type=jax.ShapeDtypeStruct((NW * NP,), jnp.float32),
    mesh=_mesh,
    compiler_params=_sc_params,
    scratch_types=[
        pltpu.VMEM((N_CHUNKS, CHUNK), jnp.int32),
        pltpu.VMEM((N_CHUNKS, CHUNK), jnp.int32),
        pltpu.VMEM((N,), jnp.float32),
        pltpu.VMEM((NP,), jnp.float32),
    ],
)
def _sc_agg2(row_hbm, col_hbm, y2_hbm, out_hbm, row_v, col_v, y2_v, acc_v):
  wid = lax.axis_index("s") * NC + lax.axis_index("c")
  pltpu.sync_copy(row_hbm.at[wid], row_v)
  pltpu.sync_copy(col_hbm.at[wid], col_v)
  pltpu.sync_copy(y2_hbm, y2_v)
  _zero_1d(acc_v, NP)

  def body(k, _):
    for t in range(CHUNK // LANES):
      r = row_v[k, pl.ds(t * LANES, LANES)]
      c = col_v[k, pl.ds(t * LANES, LANES)]
      v = plsc.load_gather(y2_v, [r])
      plsc.addupdate_scatter(acc_v, [c], v)
    return 0
  lax.fori_loop(0, N_CHUNKS, body, 0, unroll=2)
  pltpu.sync_copy(acc_v, out_hbm.at[pl.ds(wid * NP, NP)])


def _tc_d_body(p_ref, d_ref, acc_ref):
  i = pl.program_id(0)
  p = p_ref[...].reshape(1, NP)

  @pl.when(i == 0)
  def _():
    acc_ref[...] = p

  @pl.when(i > 0)
  def _():
    acc_ref[...] = acc_ref[...] + p

  @pl.when(i == NW - 1)
  def _():
    d_ref[...] = lax.rsqrt(acc_ref[:, :N] + 1.0)


def _tc_y_body(x_ref, w1_ref, d_ref, y_ref):
  xw = jnp.dot(x_ref[...], w1_ref[...], preferred_element_type=jnp.float32)
  d_col = jnp.transpose(d_ref[...], (1, 0))
  y_ref[...] = d_col * xw


def _tc_h_body(a0_ref, a1_ref, y_ref, d_ref, b1_ref, w2_ref, y2_ref):
  d_col = jnp.transpose(d_ref[...], (1, 0))
  agg = a0_ref[...] + a1_ref[...] + y_ref[...]
  h = jnp.maximum(d_col * agg + b1_ref[...], 0.0)
  hw = jnp.dot(h, w2_ref[...], preferred_element_type=jnp.float32)
  y2_ref[...] = jnp.transpose(d_col * hw, (1, 0))


def _tc_out_body(p2_ref, y2_ref, d_ref, b2_ref, o_ref, acc_ref):
  i = pl.program_id(0)
  p = p2_ref[...].reshape(1, NP)

  @pl.when(i == 0)
  def _():
    acc_ref[...] = p

  @pl.when(i > 0)
  def _():
    acc_ref[...] = acc_ref[...] + p

  @pl.when(i == NW - 1)
  def _():
    o_row = d_ref[...] * (acc_ref[:, :N] + y2_ref[...]) + b2_ref[...]
    o_ref[...] = jnp.transpose(o_row, (1, 0))


def kernel(x, edge_index, W1, b1, W2, b2):
  row3 = edge_index[0].reshape(NW, N_CHUNKS, CHUNK)
  col3 = edge_index[1].reshape(NW, N_CHUNKS, CHUNK)

  deg_part = _sc_degree(col3)

  d_row = pl.pallas_call(
      _tc_d_body,
      grid=(NW,),
      in_specs=[pl.BlockSpec((NP,), lambda i: (i,))],
      out_specs=pl.BlockSpec((1, N), lambda i: (0, 0)),
      out_shape=jax.ShapeDtypeStruct((1, N), jnp.float32),
      scratch_shapes=[pltpu.VMEM((1, NP), jnp.float32)],
  )(deg_part)

  y = pl.pallas_call(
      _tc_y_body,
      out_shape=jax.ShapeDtypeStruct((N, HID_DIM), jnp.float32),
  )(x, W1, d_row)

  agg1 = _sc_agg1(row3, col3, y)

  y2_row = pl.pallas_call(
      _tc_h_body,
      out_shape=jax.ShapeDtypeStruct((1, N), jnp.float32),
  )(agg1[0], agg1[1], y, d_row, b1.reshape(1, HID_DIM), W2)

  p2 = _sc_agg2(row3, col3, y2_row.reshape(N))

  out = pl.pallas_call(
      _tc_out_body,
      grid=(NW,),
      in_specs=[
          pl.BlockSpec((NP,), lambda i: (i,)),
          pl.BlockSpec((1, N), lambda i: (0, 0)),
          pl.BlockSpec((1, N), lambda i: (0, 0)),
          pl.BlockSpec((1, 1), lambda i: (0, 0)),
      ],
      out_specs=pl.BlockSpec((N, 1), lambda i: (0, 0)),
      out_shape=jax.ShapeDtypeStruct((N, 1), jnp.float32),
      scratch_shapes=[pltpu.VMEM((1, NP), jnp.float32)],
  )(p2, y2_row, d_row, b2.reshape(1, 1))

  return out


# speedup vs baseline: 1.1737x; 1.1737x over previous
import functools

import jax
import jax.numpy as jnp
from jax import lax
from jax.experimental import pallas as pl
from jax.experimental.pallas import tpu as pltpu
from jax.experimental.pallas import tpu_sc as plsc

N = 10000
E = 320000
IN_DIM = 128
HID_DIM = 32

NC = 2
NS = 16
NW = NC * NS
LANES = 16

E_PER_W = E // NW
CHUNK = 80
N_CHUNKS = E_PER_W // CHUNK
NBUF = 4
ROWS_PER_TILE = N // NS
NP = 10240
RROWS = NP // LANES
RPT = RROWS // NS

_mesh = plsc.VectorSubcoreMesh(core_axis_name="c", subcore_axis_name="s")
_sc_params = pltpu.CompilerParams(
    needs_layout_passes=False, use_tc_tiling_on_sc=False)


def _zero_rows(ref, nrows):
  def body(j, _):
    ref[j, :] = jnp.zeros((LANES,), ref.dtype)
    return 0
  lax.fori_loop(0, nrows, body, 0, unroll=8)


def _reduce_to_spmem_and_writeout(acc2, idx5, zbuf, dstage, sh, out_hbm,
                                  cid, sid):
  iota = lax.iota(jnp.int32, LANES)
  for j in range(RROWS // 128):
    for t in range(128 // LANES):
      idx5[j, pl.ds(t * LANES, LANES)] = iota + (128 * j + LANES * t)
  for j in range(RROWS // 128):
    pltpu.sync_copy(acc2.at[pl.ds(128 * j, 128)], sh.at[idx5.at[j]], add=True)
  plsc.subcore_barrier()
  pltpu.sync_copy(sh.at[pl.ds(sid * RPT, RPT)], zbuf)
  def cb(j, _):
    dstage[pl.ds(j * LANES, LANES)] = zbuf[j, :]
    return 0
  lax.fori_loop(0, RPT, cb, 0, unroll=8)
  pltpu.sync_copy(
      dstage, out_hbm.at[pl.ds(cid * NP + sid * RPT * LANES, RPT * LANES)])


@functools.partial(
    pl.kernel,
    out_type=jax.ShapeDtypeStruct((NC * NP,), jnp.float32),
    mesh=_mesh,
    compiler_params=_sc_params,
    scratch_types=[
        pltpu.VMEM((N_CHUNKS, CHUNK), jnp.int32),
        pltpu.VMEM((RROWS, LANES), jnp.float32),
        pltpu.VMEM((RROWS // 128, 128), jnp.int32),
        pltpu.VMEM((RPT, LANES), jnp.float32),
        pltpu.VMEM((RPT * LANES,), jnp.float32),
        pltpu.VMEM_SHARED((RROWS, LANES), jnp.float32),
    ],
)
def _sc_degree(col_hbm, out_hbm, col_v, acc2, idx5, zbuf, dstage, deg_sh):
  cid = lax.axis_index("c")
  sid = lax.axis_index("s")
  wid = sid * NC + cid
  pltpu.sync_copy(col_hbm.at[wid], col_v)
  _zero_rows(acc2, RROWS)
  _zero_rows(zbuf, RPT)
  pltpu.sync_copy(zbuf, deg_sh.at[pl.ds(sid * RPT, RPT)])
  plsc.subcore_barrier()
  ones = jnp.ones((LANES,), jnp.float32)

  def body(k, _):
    for t in range(CHUNK // LANES):
      c = col_v[k, pl.ds(t * LANES, LANES)]
      plsc.addupdate_scatter(
          acc2, [jnp.right_shift(c, 4), jnp.bitwise_and(c, 15)], ones)
    return 0
  lax.fori_loop(0, N_CHUNKS, body, 0, unroll=2)
  _reduce_to_spmem_and_writeout(
      acc2, idx5, zbuf, dstage, deg_sh, out_hbm, cid, sid)


@functools.partial(
    pl.kernel,
    out_type=jax.ShapeDtypeStruct((NC, N, HID_DIM), jnp.float32),
    mesh=_mesh,
    compiler_params=_sc_params,
    scratch_types=[
        pltpu.VMEM((N_CHUNKS, CHUNK), jnp.int32),
        pltpu.VMEM((N_CHUNKS, CHUNK), jnp.int32),
        pltpu.VMEM((NBUF, CHUNK, HID_DIM), jnp.float32),
        pltpu.VMEM((ROWS_PER_TILE, HID_DIM), jnp.float32),
        pltpu.VMEM_SHARED((N, HID_DIM), jnp.float32),
        pltpu.SemaphoreType.DMA,
    ],
)
def _sc_agg1(row_hbm, col_hbm, y_hbm, out_hbm,
             ridx_v, cidx_v, rows_v, stage_v, agg_sh, sem):
  cid = lax.axis_index("c")
  sid = lax.axis_index("s")
  wid = sid * NC + cid

  def zbody(j, _):
    stage_v[j, pl.ds(0, LANES)] = jnp.zeros((LANES,), jnp.float32)
    stage_v[j, pl.ds(LANES, LANES)] = jnp.zeros((LANES,), jnp.float32)
    return 0
  lax.fori_loop(0, ROWS_PER_TILE, zbody, 0, unroll=8)
  pltpu.sync_copy(stage_v, agg_sh.at[pl.ds(sid * ROWS_PER_TILE, ROWS_PER_TILE)])

  pltpu.sync_copy(row_hbm.at[wid], ridx_v)
  pltpu.sync_copy(col_hbm.at[wid], cidx_v)
  plsc.subcore_barrier()

  for b in range(NBUF):
    pltpu.async_copy(y_hbm.at[ridx_v.at[b]], rows_v.at[b], sem)

  def body(k, _):
    b = lax.rem(k, NBUF)
    pltpu.make_async_copy(y_hbm.at[ridx_v.at[k]], rows_v.at[b], sem).wait()
    pltpu.sync_copy(rows_v.at[b], agg_sh.at[cidx_v.at[k]], add=True)
    nk = k + NBUF

    @pl.when(nk < N_CHUNKS)
    def _():
      pltpu.async_copy(y_hbm.at[ridx_v.at[nk]], rows_v.at[b], sem)
    return 0
  lax.fori_loop(0, N_CHUNKS, body, 0)
  plsc.subcore_barrier()

  pltpu.sync_copy(agg_sh.at[pl.ds(sid * ROWS_PER_TILE, ROWS_PER_TILE)], stage_v)
  pltpu.sync_copy(stage_v, out_hbm.at[cid, pl.ds(sid * ROWS_PER_TILE, ROWS_PER_TILE)])


@functools.partial(
    pl.kernel,
    out_type=jax.ShapeDtypeStruct((NC * NP,), jnp.float32),
    mesh=_mesh,
    compiler_params=_sc_params,
    scratch_types=[
        pltpu.VMEM((N_CHUNKS, CHUNK), jnp.int32),
        pltpu.VMEM((N_CHUNKS, CHUNK), jnp.int32),
        pltpu.VMEM((N,), jnp.float32),
        pltpu.VMEM((RROWS, LANES), jnp.float32),
        pltpu.VMEM((RROWS // 128, 128), jnp.int32),
        pltpu.VMEM((RPT, LANES), jnp.float32),
        pltpu.VMEM((RPT * LANES,), jnp.float32),
        pltpu.VMEM_SHARED((RROWS, LANES), jnp.float32),
    ],
)
def _sc_agg2(row_hbm, col_hbm, y2_hbm, out_hbm,
             row_v, col_v, y2_v, acc2, idx5, zbuf, dstage, agg_sh):
  cid = lax.axis_index("c")
  sid = lax.axis_index("s")
  wid = sid * NC + cid
  pltpu.sync_copy(row_hbm.at[wid], row_v)
  pltpu.sync_copy(col_hbm.at[wid], col_v)
  pltpu.sync_copy(y2_hbm, y2_v)
  _zero_rows(acc2, RROWS)
  _zero_rows(zbuf, RPT)
  pltpu.sync_copy(zbuf, agg_sh.at[pl.ds(sid * RPT, RPT)])
  plsc.subcore_barrier()

  def body(k, _):
    for t in range(CHUNK // LANES):
      r = row_v[k, pl.ds(t * LANES, LANES)]
      c = col_v[k, pl.ds(t * LANES, LANES)]
      v = plsc.load_gather(y2_v, [r])
      plsc.addupdate_scatter(
          acc2, [jnp.right_shift(c, 4), jnp.bitwise_and(c, 15)], v)
    return 0
  lax.fori_loop(0, N_CHUNKS, body, 0, unroll=2)
  _reduce_to_spmem_and_writeout(
      acc2, idx5, zbuf, dstage, agg_sh, out_hbm, cid, sid)


def _tc_d_body(p_ref, d_ref, acc_ref):
  i = pl.program_id(0)
  p = p_ref[...].reshape(1, NP)

  @pl.when(i == 0)
  def _():
    acc_ref[...] = p

  @pl.when(i == NC - 1)
  def _():
    d_ref[...] = lax.rsqrt(acc_ref[:, :N] + p[:, :N] + 1.0)


def _tc_y_body(x_ref, w1_ref, d_ref, y_ref):
  xw = jnp.dot(x_ref[...], w1_ref[...], preferred_element_type=jnp.float32)
  d_col = jnp.transpose(d_ref[...], (1, 0))
  y_ref[...] = d_col * xw


def _tc_h_body(a0_ref, a1_ref, y_ref, d_ref, b1_ref, w2_ref, y2_ref):
  d_col = jnp.transpose(d_ref[...], (1, 0))
  agg = a0_ref[...] + a1_ref[...] + y_ref[...]
  h = jnp.maximum(d_col * agg + b1_ref[...], 0.0)
  hw = jnp.dot(h, w2_ref[...], preferred_element_type=jnp.float32)
  y2_ref[...] = jnp.transpose(d_col * hw, (1, 0))


def _tc_out_body(p2_ref, y2_ref, d_ref, b2_ref, o_ref, acc_ref):
  i = pl.program_id(0)
  p = p2_ref[...].reshape(1, NP)

  @pl.when(i == 0)
  def _():
    acc_ref[...] = p

  @pl.when(i == NC - 1)
  def _():
    o_row = (d_ref[...] * (acc_ref[:, :N] + p[:, :N] + y2_ref[...])
             + b2_ref[...])
    o_ref[...] = jnp.transpose(o_row, (1, 0))


def kernel(x, edge_index, W1, b1, W2, b2):
  row3 = edge_index[0].reshape(NW, N_CHUNKS, CHUNK)
  col3 = edge_index[1].reshape(NW, N_CHUNKS, CHUNK)

  deg_part = _sc_degree(col3)

  d_row = pl.pallas_call(
      _tc_d_body,
      grid=(NC,),
      in_specs=[pl.BlockSpec((NP,), lambda i: (i,))],
      out_specs=pl.BlockSpec((1, N), lambda i: (0, 0)),
      out_shape=jax.ShapeDtypeStruct((1, N), jnp.float32),
      scratch_shapes=[pltpu.VMEM((1, NP), jnp.float32)],
  )(deg_part)

  y = pl.pallas_call(
      _tc_y_body,
      out_shape=jax.ShapeDtypeStruct((N, HID_DIM), jnp.float32),
  )(x, W1, d_row)

  agg1 = _sc_agg1(row3, col3, y)

  y2_row = pl.pallas_call(
      _tc_h_body,
      out_shape=jax.ShapeDtypeStruct((1, N), jnp.float32),
  )(agg1[0], agg1[1], y, d_row, b1.reshape(1, HID_DIM), W2)

  p2 = _sc_agg2(row3, col3, y2_row.reshape(N))

  out = pl.pallas_call(
      _tc_out_body,
      grid=(NC,),
      in_specs=[
          pl.BlockSpec((NP,), lambda i: (i,)),
          pl.BlockSpec((1, N), lambda i: (0, 0)),
          pl.BlockSpec((1, N), lambda i: (0, 0)),
          pl.BlockSpec((1, 1), lambda i: (0, 0)),
      ],
      out_specs=pl.BlockSpec((N, 1), lambda i: (0, 0)),
      out_shape=jax.ShapeDtypeStruct((N, 1), jnp.float32),
      scratch_shapes=[pltpu.VMEM((1, NP), jnp.float32)],
  )(p2, y2_row, d_row, b2.reshape(1, 1))

  return out


# speedup vs baseline: 1.1966x; 1.0195x over previous
import functools

import jax
import jax.numpy as jnp
from jax import lax
from jax.experimental import pallas as pl
from jax.experimental.pallas import tpu as pltpu
from jax.experimental.pallas import tpu_sc as plsc

N = 10000
E = 320000
IN_DIM = 128
HID_DIM = 32

NC = 2
NS = 16
NW = NC * NS
LANES = 16

CHUNK = 128
NCH = E // CHUNK
CPT = NCH // NW
TAIL = NCH - CPT * NW
NBUF = 4
ROWS_PER_TILE = N // NS
NP = 10240
RROWS = NP // LANES
RPT = RROWS // NS

_mesh = plsc.VectorSubcoreMesh(core_axis_name="c", subcore_axis_name="s")
_sc_params = pltpu.CompilerParams(
    needs_layout_passes=False, use_tc_tiling_on_sc=False)


def _zero_rows(ref, nrows):
  def body(j, _):
    ref[j, :] = jnp.zeros((LANES,), ref.dtype)
    return 0
  lax.fori_loop(0, nrows, body, 0, unroll=8)


def _stage_idx(src2d, dst_v, wid):
  pltpu.sync_copy(src2d.at[pl.ds(CPT * wid, CPT)], dst_v.at[pl.ds(0, CPT)])

  @pl.when(wid < TAIL)
  def _():
    pltpu.sync_copy(src2d.at[pl.ds(CPT * NW + wid, 1)],
                    dst_v.at[pl.ds(CPT, 1)])


def _reduce_to_spmem_and_writeout(acc2, idx5, zbuf, dstage, sh, out_hbm,
                                  cid, sid):
  iota = lax.iota(jnp.int32, LANES)
  for j in range(RROWS // 128):
    for t in range(128 // LANES):
      idx5[j, pl.ds(t * LANES, LANES)] = iota + (128 * j + LANES * t)
  for j in range(RROWS // 128):
    pltpu.sync_copy(acc2.at[pl.ds(128 * j, 128)], sh.at[idx5.at[j]], add=True)
  plsc.subcore_barrier()
  pltpu.sync_copy(sh.at[pl.ds(sid * RPT, RPT)], zbuf)
  def cb(j, _):
    dstage[pl.ds(j * LANES, LANES)] = zbuf[j, :]
    return 0
  lax.fori_loop(0, RPT, cb, 0, unroll=8)
  pltpu.sync_copy(
      dstage, out_hbm.at[pl.ds(cid * NP + sid * RPT * LANES, RPT * LANES)])


@functools.partial(
    pl.kernel,
    out_type=jax.ShapeDtypeStruct((NC * NP,), jnp.float32),
    mesh=_mesh,
    compiler_params=_sc_params,
    scratch_types=[
        pltpu.VMEM((CPT + 1, CHUNK), jnp.int32),
        pltpu.VMEM((RROWS, LANES), jnp.float32),
        pltpu.VMEM((RROWS // 128, 128), jnp.int32),
        pltpu.VMEM((RPT, LANES), jnp.float32),
        pltpu.VMEM((RPT * LANES,), jnp.float32),
        pltpu.VMEM_SHARED((RROWS, LANES), jnp.float32),
    ],
)
def _sc_degree(col_hbm, out_hbm, col_v, acc2, idx5, zbuf, dstage, deg_sh):
  cid = lax.axis_index("c")
  sid = lax.axis_index("s")
  wid = sid * NC + cid
  _stage_idx(col_hbm, col_v, wid)
  _zero_rows(acc2, RROWS)
  _zero_rows(zbuf, RPT)
  pltpu.sync_copy(zbuf, deg_sh.at[pl.ds(sid * RPT, RPT)])
  plsc.subcore_barrier()
  ones = jnp.ones((LANES,), jnp.float32)

  def hist(k, _):
    for t in range(CHUNK // LANES):
      c = col_v[k, pl.ds(t * LANES, LANES)]
      plsc.addupdate_scatter(
          acc2, [jnp.right_shift(c, 4), jnp.bitwise_and(c, 15)], ones)
    return 0
  lax.fori_loop(0, CPT, hist, 0, unroll=2)

  @pl.when(wid < TAIL)
  def _():
    hist(CPT, 0)
  _reduce_to_spmem_and_writeout(
      acc2, idx5, zbuf, dstage, deg_sh, out_hbm, cid, sid)


@functools.partial(
    pl.kernel,
    out_type=jax.ShapeDtypeStruct((NC, N, HID_DIM), jnp.float32),
    mesh=_mesh,
    compiler_params=_sc_params,
    scratch_types=[
        pltpu.VMEM((CPT + 1, CHUNK), jnp.int32),
        pltpu.VMEM((CPT + 1, CHUNK), jnp.int32),
        pltpu.VMEM((NBUF, CHUNK, HID_DIM), jnp.float32),
        pltpu.VMEM((ROWS_PER_TILE, HID_DIM), jnp.float32),
        pltpu.VMEM_SHARED((N, HID_DIM), jnp.float32),
        pltpu.VMEM_SHARED((N, HID_DIM), jnp.float32),
        pltpu.SemaphoreType.DMA,
    ],
)
def _sc_agg1(row_hbm, col_hbm, y_hbm, out_hbm,
             ridx_v, cidx_v, rows_v, stage_v, y_sh, agg_sh, sem):
  cid = lax.axis_index("c")
  sid = lax.axis_index("s")
  wid = sid * NC + cid
  n_mine = CPT + jnp.where(wid < TAIL, 1, 0)

  pltpu.sync_copy(y_hbm.at[pl.ds(sid * ROWS_PER_TILE, ROWS_PER_TILE)],
                  y_sh.at[pl.ds(sid * ROWS_PER_TILE, ROWS_PER_TILE)])

  def zbody(j, _):
    stage_v[j, pl.ds(0, LANES)] = jnp.zeros((LANES,), jnp.float32)
    stage_v[j, pl.ds(LANES, LANES)] = jnp.zeros((LANES,), jnp.float32)
    return 0
  lax.fori_loop(0, ROWS_PER_TILE, zbody, 0, unroll=8)
  pltpu.sync_copy(stage_v, agg_sh.at[pl.ds(sid * ROWS_PER_TILE, ROWS_PER_TILE)])

  _stage_idx(row_hbm, ridx_v, wid)
  _stage_idx(col_hbm, cidx_v, wid)
  plsc.subcore_barrier()

  for b in range(NBUF):
    pltpu.async_copy(y_sh.at[ridx_v.at[b]], rows_v.at[b], sem)

  def body(k, _):
    b = lax.rem(k, NBUF)
    pltpu.make_async_copy(y_sh.at[ridx_v.at[k]], rows_v.at[b], sem).wait()
    pltpu.sync_copy(rows_v.at[b], agg_sh.at[cidx_v.at[k]], add=True)
    nk = k + NBUF

    @pl.when(nk < n_mine)
    def _():
      pltpu.async_copy(y_sh.at[ridx_v.at[nk]], rows_v.at[b], sem)
    return 0
  lax.fori_loop(0, CPT, body, 0)

  @pl.when(wid < TAIL)
  def _():
    body(CPT, 0)
  plsc.subcore_barrier()

  pltpu.sync_copy(agg_sh.at[pl.ds(sid * ROWS_PER_TILE, ROWS_PER_TILE)],
                  out_hbm.at[cid, pl.ds(sid * ROWS_PER_TILE, ROWS_PER_TILE)])


@functools.partial(
    pl.kernel,
    out_type=jax.ShapeDtypeStruct((NC * NP,), jnp.float32),
    mesh=_mesh,
    compiler_params=_sc_params,
    scratch_types=[
        pltpu.VMEM((CPT + 1, CHUNK), jnp.int32),
        pltpu.VMEM((CPT + 1, CHUNK), jnp.int32),
        pltpu.VMEM((N,), jnp.float32),
        pltpu.VMEM((RROWS, LANES), jnp.float32),
        pltpu.VMEM((RROWS // 128, 128), jnp.int32),
        pltpu.VMEM((RPT, LANES), jnp.float32),
        pltpu.VMEM((RPT * LANES,), jnp.float32),
        pltpu.VMEM_SHARED((RROWS, LANES), jnp.float32),
    ],
)
def _sc_agg2(row_hbm, col_hbm, y2_hbm, out_hbm,
             row_v, col_v, y2_v, acc2, idx5, zbuf, dstage, agg_sh):
  cid = lax.axis_index("c")
  sid = lax.axis_index("s")
  wid = sid * NC + cid
  _stage_idx(row_hbm, row_v, wid)
  _stage_idx(col_hbm, col_v, wid)
  pltpu.sync_copy(y2_hbm, y2_v)
  _zero_rows(acc2, RROWS)
  _zero_rows(zbuf, RPT)
  pltpu.sync_copy(zbuf, agg_sh.at[pl.ds(sid * RPT, RPT)])
  plsc.subcore_barrier()

  def body(k, _):
    for t in range(CHUNK // LANES):
      r = row_v[k, pl.ds(t * LANES, LANES)]
      c = col_v[k, pl.ds(t * LANES, LANES)]
      v = plsc.load_gather(y2_v, [r])
      plsc.addupdate_scatter(
          acc2, [jnp.right_shift(c, 4), jnp.bitwise_and(c, 15)], v)
    return 0
  lax.fori_loop(0, CPT, body, 0, unroll=2)

  @pl.when(wid < TAIL)
  def _():
    body(CPT, 0)
  _reduce_to_spmem_and_writeout(
      acc2, idx5, zbuf, dstage, agg_sh, out_hbm, cid, sid)


def _tc_d_body(p_ref, d_ref, acc_ref):
  i = pl.program_id(0)
  p = p_ref[...].reshape(1, NP)

  @pl.when(i == 0)
  def _():
    acc_ref[...] = p

  @pl.when(i == NC - 1)
  def _():
    d_ref[...] = lax.rsqrt(acc_ref[:, :N] + p[:, :N] + 1.0)


def _tc_y_body(x_ref, w1_ref, d_ref, y_ref):
  xw = jnp.dot(x_ref[...], w1_ref[...], preferred_element_type=jnp.float32)
  d_col = jnp.transpose(d_ref[...], (1, 0))
  y_ref[...] = d_col * xw


def _tc_h_body(a_ref, y_ref, d_ref, b1_ref, w2_ref, y2_ref):
  d_col = jnp.transpose(d_ref[...], (1, 0))
  agg = a_ref[0] + a_ref[1] + y_ref[...]
  h = jnp.maximum(d_col * agg + b1_ref[...], 0.0)
  hw = jnp.dot(h, w2_ref[...], preferred_element_type=jnp.float32)
  y2_ref[...] = jnp.transpose(d_col * hw, (1, 0))


def _tc_out_body(p2_ref, y2_ref, d_ref, b2_ref, o_ref, acc_ref):
  i = pl.program_id(0)
  p = p2_ref[...].reshape(1, NP)

  @pl.when(i == 0)
  def _():
    acc_ref[...] = p

  @pl.when(i == NC - 1)
  def _():
    o_row = (d_ref[...] * (acc_ref[:, :N] + p[:, :N] + y2_ref[...])
             + b2_ref[...])
    o_ref[...] = jnp.transpose(o_row, (1, 0))


def kernel(x, edge_index, W1, b1, W2, b2):
  row2 = edge_index[0].reshape(NCH, CHUNK)
  col2 = edge_index[1].reshape(NCH, CHUNK)

  deg_part = _sc_degree(col2)

  d_row = pl.pallas_call(
      _tc_d_body,
      grid=(NC,),
      in_specs=[pl.BlockSpec((NP,), lambda i: (i,))],
      out_specs=pl.BlockSpec((1, N), lambda i: (0, 0)),
      out_shape=jax.ShapeDtypeStruct((1, N), jnp.float32),
      scratch_shapes=[pltpu.VMEM((1, NP), jnp.float32)],
  )(deg_part)

  y = pl.pallas_call(
      _tc_y_body,
      out_shape=jax.ShapeDtypeStruct((N, HID_DIM), jnp.float32),
  )(x, W1, d_row)

  agg1 = _sc_agg1(row2, col2, y)

  y2_row = pl.pallas_call(
      _tc_h_body,
      out_shape=jax.ShapeDtypeStruct((1, N), jnp.float32),
  )(agg1, y, d_row, b1.reshape(1, HID_DIM), W2)

  p2 = _sc_agg2(row2, col2, y2_row.reshape(N))

  out = pl.pallas_call(
      _tc_out_body,
      grid=(NC,),
      in_specs=[
          pl.BlockSpec((NP,), lambda i: (i,)),
          pl.BlockSpec((1, N), lambda i: (0, 0)),
          pl.BlockSpec((1, N), lambda i: (0, 0)),
          pl.BlockSpec((1, 1), lambda i: (0, 0)),
      ],
      out_specs=pl.BlockSpec((N, 1), lambda i: (0, 0)),
      out_shape=jax.ShapeDtypeStruct((N, 1), jnp.float32),
      scratch_shapes=[pltpu.VMEM((1, NP), jnp.float32)],
  )(p2, y2_row, d_row, b2.reshape(1, 1))

  return out


# speedup vs baseline: 1.2739x; 1.0646x over previous
import functools

import jax
import jax.numpy as jnp
from jax import lax
from jax.experimental import pallas as pl
from jax.experimental.pallas import tpu as pltpu
from jax.experimental.pallas import tpu_sc as plsc

N = 10000
E = 320000
IN_DIM = 128
HID_DIM = 32

NC = 2
NS = 16
NW = NC * NS
LANES = 16

CHUNK = 128
NCH = E // CHUNK
CPT = NCH // NW
TAIL = NCH - CPT * NW
NBUF = 4
ROWS_PER_TILE = N // NS
NP = 10240
RROWS = NP // LANES
RPT = RROWS // NS

_mesh = plsc.VectorSubcoreMesh(core_axis_name="c", subcore_axis_name="s")
_sc_params = pltpu.CompilerParams(
    needs_layout_passes=False, use_tc_tiling_on_sc=False)


def _zero_rows(ref, nrows):
  def body(j, _):
    ref[j, :] = jnp.zeros((LANES,), ref.dtype)
    return 0
  lax.fori_loop(0, nrows, body, 0, unroll=8)


def _stage_idx(src2d, dst_v, wid):
  pltpu.sync_copy(src2d.at[pl.ds(CPT * wid, CPT)], dst_v.at[pl.ds(0, CPT)])

  @pl.when(wid < TAIL)
  def _():
    pltpu.sync_copy(src2d.at[pl.ds(CPT * NW + wid, 1)],
                    dst_v.at[pl.ds(CPT, 1)])


def _reduce_to_spmem_and_writeout(acc2, idx5, zbuf, dstage, sh, out_hbm,
                                  cid, sid):
  iota = lax.iota(jnp.int32, LANES)
  for j in range(RROWS // 128):
    for t in range(128 // LANES):
      idx5[j, pl.ds(t * LANES, LANES)] = iota + (128 * j + LANES * t)
  for j in range(RROWS // 128):
    pltpu.sync_copy(acc2.at[pl.ds(128 * j, 128)], sh.at[idx5.at[j]], add=True)
  plsc.subcore_barrier()
  pltpu.sync_copy(sh.at[pl.ds(sid * RPT, RPT)], zbuf)
  def cb(j, _):
    dstage[pl.ds(j * LANES, LANES)] = zbuf[j, :]
    return 0
  lax.fori_loop(0, RPT, cb, 0, unroll=8)
  pltpu.sync_copy(
      dstage, out_hbm.at[pl.ds(cid * NP + sid * RPT * LANES, RPT * LANES)])


@functools.partial(
    pl.kernel,
    out_type=jax.ShapeDtypeStruct((NC * NP,), jnp.float32),
    mesh=_mesh,
    compiler_params=_sc_params,
    scratch_types=[
        pltpu.VMEM((CPT + 1, CHUNK), jnp.int32),
        pltpu.VMEM((RROWS, LANES), jnp.float32),
        pltpu.VMEM((RROWS // 128, 128), jnp.int32),
        pltpu.VMEM((RPT, LANES), jnp.float32),
        pltpu.VMEM((RPT * LANES,), jnp.float32),
        pltpu.VMEM_SHARED((RROWS, LANES), jnp.float32),
    ],
)
def _sc_degree(col_hbm, out_hbm, col_v, acc2, idx5, zbuf, dstage, deg_sh):
  cid = lax.axis_index("c")
  sid = lax.axis_index("s")
  wid = sid * NC + cid
  _stage_idx(col_hbm, col_v, wid)
  _zero_rows(acc2, RROWS)
  _zero_rows(zbuf, RPT)
  pltpu.sync_copy(zbuf, deg_sh.at[pl.ds(sid * RPT, RPT)])
  plsc.subcore_barrier()
  ones = jnp.ones((LANES,), jnp.float32)

  def hist(k, _):
    for t in range(CHUNK // LANES):
      c = col_v[k, pl.ds(t * LANES, LANES)]
      plsc.addupdate_scatter(
          acc2, [jnp.right_shift(c, 4), jnp.bitwise_and(c, 15)], ones)
    return 0
  lax.fori_loop(0, CPT, hist, 0, unroll=2)

  @pl.when(wid < TAIL)
  def _():
    hist(CPT, 0)
  _reduce_to_spmem_and_writeout(
      acc2, idx5, zbuf, dstage, deg_sh, out_hbm, cid, sid)


@functools.partial(
    pl.kernel,
    out_type=jax.ShapeDtypeStruct((NC, N, HID_DIM), jnp.float32),
    mesh=_mesh,
    compiler_params=_sc_params,
    scratch_types=[
        pltpu.VMEM((CPT + 1, CHUNK), jnp.int32),
        pltpu.VMEM((CPT + 1, CHUNK), jnp.int32),
        pltpu.VMEM((NBUF, CHUNK, HID_DIM), jnp.float32),
        pltpu.VMEM((ROWS_PER_TILE, HID_DIM), jnp.float32),
        pltpu.VMEM_SHARED((N, HID_DIM), jnp.float32),
        pltpu.SemaphoreType.DMA,
    ],
)
def _sc_agg1(row_hbm, col_hbm, y_hbm, out_hbm,
             ridx_v, cidx_v, rows_v, stage_v, agg_sh, sem):
  cid = lax.axis_index("c")
  sid = lax.axis_index("s")
  wid = sid * NC + cid
  n_mine = CPT + jnp.where(wid < TAIL, 1, 0)

  def zbody(j, _):
    stage_v[j, pl.ds(0, LANES)] = jnp.zeros((LANES,), jnp.float32)
    stage_v[j, pl.ds(LANES, LANES)] = jnp.zeros((LANES,), jnp.float32)
    return 0
  lax.fori_loop(0, ROWS_PER_TILE, zbody, 0, unroll=8)
  pltpu.sync_copy(stage_v, agg_sh.at[pl.ds(sid * ROWS_PER_TILE, ROWS_PER_TILE)])

  _stage_idx(row_hbm, ridx_v, wid)
  _stage_idx(col_hbm, cidx_v, wid)
  plsc.subcore_barrier()

  for b in range(NBUF):
    pltpu.async_copy(y_hbm.at[ridx_v.at[b]], rows_v.at[b], sem)

  def body(k, _):
    b = lax.rem(k, NBUF)
    pltpu.make_async_copy(y_hbm.at[ridx_v.at[k]], rows_v.at[b], sem).wait()
    pltpu.sync_copy(rows_v.at[b], agg_sh.at[cidx_v.at[k]], add=True)
    nk = k + NBUF

    @pl.when(nk < n_mine)
    def _():
      pltpu.async_copy(y_hbm.at[ridx_v.at[nk]], rows_v.at[b], sem)
    return 0
  lax.fori_loop(0, CPT, body, 0)

  @pl.when(wid < TAIL)
  def _():
    body(CPT, 0)
  plsc.subcore_barrier()

  pltpu.sync_copy(agg_sh.at[pl.ds(sid * ROWS_PER_TILE, ROWS_PER_TILE)],
                  out_hbm.at[cid, pl.ds(sid * ROWS_PER_TILE, ROWS_PER_TILE)])


@functools.partial(
    pl.kernel,
    out_type=jax.ShapeDtypeStruct((NC * NP,), jnp.float32),
    mesh=_mesh,
    compiler_params=_sc_params,
    scratch_types=[
        pltpu.VMEM((CPT + 1, CHUNK), jnp.int32),
        pltpu.VMEM((CPT + 1, CHUNK), jnp.int32),
        pltpu.VMEM((N,), jnp.float32),
        pltpu.VMEM((RROWS, LANES), jnp.float32),
        pltpu.VMEM((RROWS // 128, 128), jnp.int32),
        pltpu.VMEM((RPT, LANES), jnp.float32),
        pltpu.VMEM((RPT * LANES,), jnp.float32),
        pltpu.VMEM_SHARED((RROWS, LANES), jnp.float32),
    ],
)
def _sc_agg2(row_hbm, col_hbm, y2_hbm, out_hbm,
             row_v, col_v, y2_v, acc2, idx5, zbuf, dstage, agg_sh):
  cid = lax.axis_index("c")
  sid = lax.axis_index("s")
  wid = sid * NC + cid
  _stage_idx(row_hbm, row_v, wid)
  _stage_idx(col_hbm, col_v, wid)
  pltpu.sync_copy(y2_hbm, y2_v)
  _zero_rows(acc2, RROWS)
  _zero_rows(zbuf, RPT)
  pltpu.sync_copy(zbuf, agg_sh.at[pl.ds(sid * RPT, RPT)])
  plsc.subcore_barrier()

  def body(k, _):
    for t in range(CHUNK // LANES):
      r = row_v[k, pl.ds(t * LANES, LANES)]
      c = col_v[k, pl.ds(t * LANES, LANES)]
      v = plsc.load_gather(y2_v, [r])
      plsc.addupdate_scatter(
          acc2, [jnp.right_shift(c, 4), jnp.bitwise_and(c, 15)], v)
    return 0
  lax.fori_loop(0, CPT, body, 0, unroll=2)

  @pl.when(wid < TAIL)
  def _():
    body(CPT, 0)
  _reduce_to_spmem_and_writeout(
      acc2, idx5, zbuf, dstage, agg_sh, out_hbm, cid, sid)


def _tc_d_body(p_ref, d_ref, acc_ref):
  i = pl.program_id(0)
  p = p_ref[...].reshape(1, NP)

  @pl.when(i == 0)
  def _():
    acc_ref[...] = p

  @pl.when(i == NC - 1)
  def _():
    d_ref[...] = lax.rsqrt(acc_ref[:, :N] + p[:, :N] + 1.0)


def _tc_y_body(x_ref, w1_ref, d_ref, y_ref):
  xw = jnp.dot(x_ref[...], w1_ref[...], preferred_element_type=jnp.float32)
  d_col = jnp.transpose(d_ref[...], (1, 0))
  y_ref[...] = d_col * xw


def _tc_h_body(a_ref, y_ref, d_ref, b1_ref, w2_ref, y2_ref):
  d_col = jnp.transpose(d_ref[...], (1, 0))
  agg = a_ref[0] + a_ref[1] + y_ref[...]
  h = jnp.maximum(d_col * agg + b1_ref[...], 0.0)
  hw = jnp.dot(h, w2_ref[...], preferred_element_type=jnp.float32)
  y2_ref[...] = jnp.transpose(d_col * hw, (1, 0))


def _tc_out_body(p2_ref, y2_ref, d_ref, b2_ref, o_ref, acc_ref):
  i = pl.program_id(0)
  p = p2_ref[...].reshape(1, NP)

  @pl.when(i == 0)
  def _():
    acc_ref[...] = p

  @pl.when(i == NC - 1)
  def _():
    o_row = (d_ref[...] * (acc_ref[:, :N] + p[:, :N] + y2_ref[...])
             + b2_ref[...])
    o_ref[...] = jnp.transpose(o_row, (1, 0))


def kernel(x, edge_index, W1, b1, W2, b2):
  row2 = edge_index[0].reshape(NCH, CHUNK)
  col2 = edge_index[1].reshape(NCH, CHUNK)

  deg_part = _sc_degree(col2)

  d_row = pl.pallas_call(
      _tc_d_body,
      grid=(NC,),
      in_specs=[pl.BlockSpec((NP,), lambda i: (i,))],
      out_specs=pl.BlockSpec((1, N), lambda i: (0, 0)),
      out_shape=jax.ShapeDtypeStruct((1, N), jnp.float32),
      scratch_shapes=[pltpu.VMEM((1, NP), jnp.float32)],
  )(deg_part)

  y = pl.pallas_call(
      _tc_y_body,
      out_shape=jax.ShapeDtypeStruct((N, HID_DIM), jnp.float32),
  )(x, W1, d_row)

  agg1 = _sc_agg1(row2, col2, y)

  y2_row = pl.pallas_call(
      _tc_h_body,
      out_shape=jax.ShapeDtypeStruct((1, N), jnp.float32),
  )(agg1, y, d_row, b1.reshape(1, HID_DIM), W2)

  p2 = _sc_agg2(row2, col2, y2_row.reshape(N))

  out = pl.pallas_call(
      _tc_out_body,
      grid=(NC,),
      in_specs=[
          pl.BlockSpec((NP,), lambda i: (i,)),
          pl.BlockSpec((1, N), lambda i: (0, 0)),
          pl.BlockSpec((1, N), lambda i: (0, 0)),
          pl.BlockSpec((1, 1), lambda i: (0, 0)),
      ],
      out_specs=pl.BlockSpec((N, 1), lambda i: (0, 0)),
      out_shape=jax.ShapeDtypeStruct((N, 1), jnp.float32),
      scratch_shapes=[pltpu.VMEM((1, NP), jnp.float32)],
  )(p2, y2_row, d_row, b2.reshape(1, 1))

  return out


# speedup vs baseline: 1.2834x; 1.0075x over previous
import functools

import jax
import jax.numpy as jnp
from jax import lax
from jax.experimental import pallas as pl
from jax.experimental.pallas import tpu as pltpu
from jax.experimental.pallas import tpu_sc as plsc

N = 10000
E = 320000
IN_DIM = 128
HID_DIM = 32

NC = 2
NS = 16
NW = NC * NS
LANES = 16

CHUNK = 128
NCH = E // CHUNK
CPT = NCH // NW
TAIL = NCH - CPT * NW
NBUF = 6
ROWS_PER_TILE = N // NS
NP = 10240
RROWS = NP // LANES
RPT = RROWS // NS

_mesh = plsc.VectorSubcoreMesh(core_axis_name="c", subcore_axis_name="s")
_sc_params = pltpu.CompilerParams(
    needs_layout_passes=False, use_tc_tiling_on_sc=False)


def _zero_rows(ref, nrows):
  def body(j, _):
    ref[j, :] = jnp.zeros((LANES,), ref.dtype)
    return 0
  lax.fori_loop(0, nrows, body, 0, unroll=8)


def _stage_idx(src2d, dst_v, wid):
  pltpu.sync_copy(src2d.at[pl.ds(CPT * wid, CPT)], dst_v.at[pl.ds(0, CPT)])

  @pl.when(wid < TAIL)
  def _():
    pltpu.sync_copy(src2d.at[pl.ds(CPT * NW + wid, 1)],
                    dst_v.at[pl.ds(CPT, 1)])


def _reduce_to_spmem_and_writeout(acc2, idx5, zbuf, dstage, sh, out_hbm,
                                  cid, sid):
  iota = lax.iota(jnp.int32, LANES)
  for j in range(RROWS // 128):
    for t in range(128 // LANES):
      idx5[j, pl.ds(t * LANES, LANES)] = iota + (128 * j + LANES * t)
  for j in range(RROWS // 128):
    pltpu.sync_copy(acc2.at[pl.ds(128 * j, 128)], sh.at[idx5.at[j]], add=True)
  plsc.subcore_barrier()
  pltpu.sync_copy(sh.at[pl.ds(sid * RPT, RPT)], zbuf)
  def cb(j, _):
    dstage[pl.ds(j * LANES, LANES)] = zbuf[j, :]
    return 0
  lax.fori_loop(0, RPT, cb, 0, unroll=8)
  pltpu.sync_copy(
      dstage, out_hbm.at[pl.ds(cid * NP + sid * RPT * LANES, RPT * LANES)])


@functools.partial(
    pl.kernel,
    out_type=jax.ShapeDtypeStruct((NC * NP,), jnp.float32),
    mesh=_mesh,
    compiler_params=_sc_params,
    scratch_types=[
        pltpu.VMEM((CPT + 1, CHUNK), jnp.int32),
        pltpu.VMEM((RROWS, LANES), jnp.float32),
        pltpu.VMEM((RROWS // 128, 128), jnp.int32),
        pltpu.VMEM((RPT, LANES), jnp.float32),
        pltpu.VMEM((RPT * LANES,), jnp.float32),
        pltpu.VMEM_SHARED((RROWS, LANES), jnp.float32),
    ],
)
def _sc_degree(col_hbm, out_hbm, col_v, acc2, idx5, zbuf, dstage, deg_sh):
  cid = lax.axis_index("c")
  sid = lax.axis_index("s")
  wid = sid * NC + cid
  _stage_idx(col_hbm, col_v, wid)
  _zero_rows(acc2, RROWS)
  _zero_rows(zbuf, RPT)
  pltpu.sync_copy(zbuf, deg_sh.at[pl.ds(sid * RPT, RPT)])
  plsc.subcore_barrier()
  ones = jnp.ones((LANES,), jnp.float32)

  def hist(k, _):
    for t in range(CHUNK // LANES):
      c = col_v[k, pl.ds(t * LANES, LANES)]
      plsc.addupdate_scatter(
          acc2, [jnp.right_shift(c, 4), jnp.bitwise_and(c, 15)], ones)
    return 0
  lax.fori_loop(0, CPT, hist, 0, unroll=4)

  @pl.when(wid < TAIL)
  def _():
    hist(CPT, 0)
  _reduce_to_spmem_and_writeout(
      acc2, idx5, zbuf, dstage, deg_sh, out_hbm, cid, sid)


@functools.partial(
    pl.kernel,
    out_type=jax.ShapeDtypeStruct((NC, N, HID_DIM), jnp.float32),
    mesh=_mesh,
    compiler_params=_sc_params,
    scratch_types=[
        pltpu.VMEM((CPT + 1, CHUNK), jnp.int32),
        pltpu.VMEM((CPT + 1, CHUNK), jnp.int32),
        pltpu.VMEM((NBUF, CHUNK, HID_DIM), jnp.float32),
        pltpu.VMEM((ROWS_PER_TILE, HID_DIM), jnp.float32),
        pltpu.VMEM_SHARED((N, HID_DIM), jnp.float32),
        pltpu.SemaphoreType.DMA,
        pltpu.SemaphoreType.DMA,
    ],
)
def _sc_agg1(row_hbm, col_hbm, y_hbm, out_hbm,
             ridx_v, cidx_v, rows_v, stage_v, agg_sh, gsem, ssem):
  cid = lax.axis_index("c")
  sid = lax.axis_index("s")
  wid = sid * NC + cid
  n_mine = CPT + jnp.where(wid < TAIL, 1, 0)

  def zbody(j, _):
    stage_v[j, pl.ds(0, LANES)] = jnp.zeros((LANES,), jnp.float32)
    stage_v[j, pl.ds(LANES, LANES)] = jnp.zeros((LANES,), jnp.float32)
    return 0
  lax.fori_loop(0, ROWS_PER_TILE, zbody, 0, unroll=8)
  pltpu.sync_copy(stage_v, agg_sh.at[pl.ds(sid * ROWS_PER_TILE, ROWS_PER_TILE)])

  _stage_idx(row_hbm, ridx_v, wid)
  _stage_idx(col_hbm, cidx_v, wid)
  plsc.subcore_barrier()

  for b in range(4):
    pltpu.async_copy(y_hbm.at[ridx_v.at[b]], rows_v.at[b], gsem)

  def body(k, _):
    b = lax.rem(k, NBUF)
    pltpu.make_async_copy(y_hbm.at[ridx_v.at[k]], rows_v.at[b], gsem).wait()
    pltpu.async_copy(rows_v.at[b], agg_sh.at[cidx_v.at[k]], ssem, add=True)
    km2 = k - 2

    @pl.when(km2 >= 0)
    def _():
      pltpu.make_async_copy(rows_v.at[lax.rem(km2, NBUF)],
                            agg_sh.at[cidx_v.at[km2]], ssem).wait()
    nk = k + 4

    @pl.when(nk < n_mine)
    def _():
      pltpu.async_copy(y_hbm.at[ridx_v.at[nk]], rows_v.at[lax.rem(nk, NBUF)],
                       gsem)
    return 0
  lax.fori_loop(0, n_mine, body, 0)

  for j in (2, 1):
    k = n_mine - j

    @pl.when(k >= 0)
    def _():
      pltpu.make_async_copy(rows_v.at[lax.rem(k, NBUF)],
                            agg_sh.at[cidx_v.at[k]], ssem).wait()
  plsc.subcore_barrier()

  pltpu.sync_copy(agg_sh.at[pl.ds(sid * ROWS_PER_TILE, ROWS_PER_TILE)],
                  out_hbm.at[cid, pl.ds(sid * ROWS_PER_TILE, ROWS_PER_TILE)])


@functools.partial(
    pl.kernel,
    out_type=jax.ShapeDtypeStruct((NC * NP,), jnp.float32),
    mesh=_mesh,
    compiler_params=_sc_params,
    scratch_types=[
        pltpu.VMEM((CPT + 1, CHUNK), jnp.int32),
        pltpu.VMEM((CPT + 1, CHUNK), jnp.int32),
        pltpu.VMEM((N,), jnp.float32),
        pltpu.VMEM((RROWS, LANES), jnp.float32),
        pltpu.VMEM((RROWS // 128, 128), jnp.int32),
        pltpu.VMEM((RPT, LANES), jnp.float32),
        pltpu.VMEM((RPT * LANES,), jnp.float32),
        pltpu.VMEM_SHARED((RROWS, LANES), jnp.float32),
    ],
)
def _sc_agg2(row_hbm, col_hbm, y2_hbm, out_hbm,
             row_v, col_v, y2_v, acc2, idx5, zbuf, dstage, agg_sh):
  cid = lax.axis_index("c")
  sid = lax.axis_index("s")
  wid = sid * NC + cid
  _stage_idx(row_hbm, row_v, wid)
  _stage_idx(col_hbm, col_v, wid)
  pltpu.sync_copy(y2_hbm, y2_v)
  _zero_rows(acc2, RROWS)
  _zero_rows(zbuf, RPT)
  pltpu.sync_copy(zbuf, agg_sh.at[pl.ds(sid * RPT, RPT)])
  plsc.subcore_barrier()

  def body(k, _):
    for t in range(CHUNK // LANES):
      r = row_v[k, pl.ds(t * LANES, LANES)]
      c = col_v[k, pl.ds(t * LANES, LANES)]
      v = plsc.load_gather(y2_v, [r])
      plsc.addupdate_scatter(
          acc2, [jnp.right_shift(c, 4), jnp.bitwise_and(c, 15)], v)
    return 0
  lax.fori_loop(0, CPT, body, 0, unroll=4)

  @pl.when(wid < TAIL)
  def _():
    body(CPT, 0)
  _reduce_to_spmem_and_writeout(
      acc2, idx5, zbuf, dstage, agg_sh, out_hbm, cid, sid)


def _tc_d_body(p_ref, d_ref, acc_ref):
  i = pl.program_id(0)
  p = p_ref[...].reshape(1, NP)

  @pl.when(i == 0)
  def _():
    acc_ref[...] = p

  @pl.when(i == NC - 1)
  def _():
    d_ref[...] = lax.rsqrt(acc_ref[:, :N] + p[:, :N] + 1.0)


def _tc_y_body(x_ref, w1_ref, d_ref, y_ref):
  xw = jnp.dot(x_ref[...], w1_ref[...], preferred_element_type=jnp.float32)
  d_col = jnp.transpose(d_ref[...], (1, 0))
  y_ref[...] = d_col * xw


def _tc_h_body(a_ref, y_ref, d_ref, b1_ref, w2_ref, y2_ref):
  d_col = jnp.transpose(d_ref[...], (1, 0))
  agg = a_ref[0] + a_ref[1] + y_ref[...]
  h = jnp.maximum(d_col * agg + b1_ref[...], 0.0)
  hw = jnp.dot(h, w2_ref[...], preferred_element_type=jnp.float32)
  y2_ref[...] = jnp.transpose(d_col * hw, (1, 0))


def _tc_out_body(p2_ref, y2_ref, d_ref, b2_ref, o_ref, acc_ref):
  i = pl.program_id(0)
  p = p2_ref[...].reshape(1, NP)

  @pl.when(i == 0)
  def _():
    acc_ref[...] = p

  @pl.when(i == NC - 1)
  def _():
    o_row = (d_ref[...] * (acc_ref[:, :N] + p[:, :N] + y2_ref[...])
             + b2_ref[...])
    o_ref[...] = jnp.transpose(o_row, (1, 0))


def kernel(x, edge_index, W1, b1, W2, b2):
  row2 = edge_index[0].reshape(NCH, CHUNK)
  col2 = edge_index[1].reshape(NCH, CHUNK)

  deg_part = _sc_degree(col2)

  d_row = pl.pallas_call(
      _tc_d_body,
      grid=(NC,),
      in_specs=[pl.BlockSpec((NP,), lambda i: (i,))],
      out_specs=pl.BlockSpec((1, N), lambda i: (0, 0)),
      out_shape=jax.ShapeDtypeStruct((1, N), jnp.float32),
      scratch_shapes=[pltpu.VMEM((1, NP), jnp.float32)],
  )(deg_part)

  y = pl.pallas_call(
      _tc_y_body,
      out_shape=jax.ShapeDtypeStruct((N, HID_DIM), jnp.float32),
  )(x, W1, d_row)

  agg1 = _sc_agg1(row2, col2, y)

  y2_row = pl.pallas_call(
      _tc_h_body,
      out_shape=jax.ShapeDtypeStruct((1, N), jnp.float32),
  )(agg1, y, d_row, b1.reshape(1, HID_DIM), W2)

  p2 = _sc_agg2(row2, col2, y2_row.reshape(N))

  out = pl.pallas_call(
      _tc_out_body,
      grid=(NC,),
      in_specs=[
          pl.BlockSpec((NP,), lambda i: (i,)),
          pl.BlockSpec((1, N), lambda i: (0, 0)),
          pl.BlockSpec((1, N), lambda i: (0, 0)),
          pl.BlockSpec((1, 1), lambda i: (0, 0)),
      ],
      out_specs=pl.BlockSpec((N, 1), lambda i: (0, 0)),
      out_shape=jax.ShapeDtypeStruct((N, 1), jnp.float32),
      scratch_shapes=[pltpu.VMEM((1, NP), jnp.float32)],
  )(p2, y2_row, d_row, b2.reshape(1, 1))

  return out
